# baseline (device time: 10739 ns/iter reference)
import functools

import jax
import jax.numpy as jnp
from jax import lax
from jax.experimental import pallas as pl
from jax.experimental.pallas import tpu as pltpu

F = 160
NF = 10
ND = 2


def kernel(x):
    m_per, n = x.shape
    kf = F // NF
    d_rows = m_per - 2 * F
    kd = d_rows // ND

    def body(x_hbm, out_ref, xv, cpa_sem, cpb_sem, cpc_sem,
             p1f_send, p1f_recv, p1d_send, p1d_recv, p2_send, p2_recv):
        my_x = lax.axis_index("x")
        my_y = lax.axis_index("y")
        my_z = lax.axis_index("z")
        h = lax.rem(my_z, 2)
        px = (1 - my_x, my_y, my_z)
        pz = (my_x, my_y, my_z + 1 - 2 * h)

        own = my_x * m_per
        opp = (1 - my_x) * m_per
        fwd = (1 - h) * F
        kept = h * F

        @functools.partial(
            pl.run_scoped, pz_sem=pltpu.SemaphoreType.REGULAR
        )
        def _(pz_sem):
            barrier_sem = pltpu.get_barrier_semaphore()
            pl.semaphore_signal(
                barrier_sem,
                inc=1,
                device_id=px,
                device_id_type=pl.DeviceIdType.MESH,
            )
            pl.semaphore_signal(
                pz_sem,
                inc=1,
                device_id=pz,
                device_id_type=pl.DeviceIdType.MESH,
            )

            cpa = pltpu.make_async_copy(
                x_hbm.at[pl.ds(fwd, F), :], xv.at[pl.ds(fwd, F), :], cpa_sem
            )
            cpb = pltpu.make_async_copy(
                x_hbm.at[pl.ds(2 * F, d_rows), :],
                xv.at[pl.ds(2 * F, d_rows), :],
                cpb_sem,
            )
            cpc = pltpu.make_async_copy(
                x_hbm.at[pl.ds(kept, F), :], xv.at[pl.ds(kept, F), :], cpc_sem
            )
            cpa.start()
            cpb.start()
            cpc.start()
            pl.semaphore_wait(barrier_sem, 1)

            cpa.wait()
            p1f = []
            for c in range(NF):
                s = own + fwd + c * kf
                out_ref[pl.ds(s, kf), :] = xv[
                    pl.ds(fwd + c * kf, kf), :
                ].astype(jnp.bfloat16)
                r = pltpu.make_async_remote_copy(
                    src_ref=out_ref.at[pl.ds(s, kf), :],
                    dst_ref=out_ref.at[pl.ds(s, kf), :],
                    send_sem=p1f_send.at[c],
                    recv_sem=p1f_recv.at[c],
                    device_id=px,
                    device_id_type=pl.DeviceIdType.MESH,
                )
                r.start()
                p1f.append(r)
            cpb.wait()
            p1d = []
            for c in range(ND):
                s = own + 2 * F + c * kd
                out_ref[pl.ds(s, kd), :] = xv[
                    pl.ds(2 * F + c * kd, kd), :
                ].astype(jnp.bfloat16)
                r = pltpu.make_async_remote_copy(
                    src_ref=out_ref.at[pl.ds(s, kd), :],
                    dst_ref=out_ref.at[pl.ds(s, kd), :],
                    send_sem=p1d_send.at[c],
                    recv_sem=p1d_recv.at[c],
                    device_id=px,
                    device_id_type=pl.DeviceIdType.MESH,
                )
                r.start()
                p1d.append(r)

            cpc.wait()
            out_ref[pl.ds(own + kept, F), :] = xv[
                pl.ds(kept, F), :
            ].astype(jnp.bfloat16)

            pl.semaphore_wait(pz_sem, 1)

            p2 = []
            for c in range(NF):
                p1f[c].wait_recv()
                s = opp + fwd + c * kf
                r = pltpu.make_async_remote_copy(
                    src_ref=out_ref.at[pl.ds(s, kf), :],
                    dst_ref=out_ref.at[pl.ds(s, kf), :],
                    send_sem=p2_send.at[c],
                    recv_sem=p2_recv.at[c],
                    device_id=pz,
                    device_id_type=pl.DeviceIdType.MESH,
                )
                r.start()
                p2.append(r)

            for c in range(ND):
                p1d[c].wait_recv()
            for c in range(NF):
                p2[c].wait_recv()
                p1f[c].wait_send()
                p2[c].wait_send()
            for c in range(ND):
                p1d[c].wait_send()

    return pl.pallas_call(
        body,
        out_shape=jax.ShapeDtypeStruct((2 * m_per, n), jnp.bfloat16),
        in_specs=[pl.BlockSpec(memory_space=pl.ANY)],
        out_specs=pl.BlockSpec(memory_space=pltpu.VMEM),
        scratch_shapes=[
            pltpu.VMEM((m_per, n), x.dtype),
            pltpu.SemaphoreType.DMA,
            pltpu.SemaphoreType.DMA,
            pltpu.SemaphoreType.DMA,
            pltpu.SemaphoreType.DMA((NF,)),
            pltpu.SemaphoreType.DMA((NF,)),
            pltpu.SemaphoreType.DMA((ND,)),
            pltpu.SemaphoreType.DMA((ND,)),
            pltpu.SemaphoreType.DMA((NF,)),
            pltpu.SemaphoreType.DMA((NF,)),
        ],
        compiler_params=pltpu.CompilerParams(collective_id=0),
    )(x)


# device time: 10351 ns/iter; 1.0375x vs baseline; 1.0375x over previous
import functools

import jax
import jax.numpy as jnp
from jax import lax
from jax.experimental import pallas as pl
from jax.experimental.pallas import tpu as pltpu

F = 128
NF = 4
ND = 2


def kernel(x):
    m_per, n = x.shape
    kf = F // NF
    d_rows = m_per - 2 * F
    kd = d_rows // ND

    def body(x_hbm, out_ref, xv, cpa_sem, cpb_sem, cpc_sem,
             p1f_send, p1f_recv, p1d_send, p1d_recv, p2_send, p2_recv):
        my_x = lax.axis_index("x")
        my_y = lax.axis_index("y")
        my_z = lax.axis_index("z")
        h = lax.rem(my_z, 2)
        px = (1 - my_x, my_y, my_z)
        pz = (my_x, my_y, my_z + 1 - 2 * h)

        own = my_x * m_per
        opp = (1 - my_x) * m_per
        fwd = (1 - h) * F
        kept = h * F

        @functools.partial(
            pl.run_scoped, pz_sem=pltpu.SemaphoreType.REGULAR
        )
        def _(pz_sem):
            barrier_sem = pltpu.get_barrier_semaphore()
            pl.semaphore_signal(
                barrier_sem,
                inc=1,
                device_id=px,
                device_id_type=pl.DeviceIdType.MESH,
            )
            pl.semaphore_signal(
                pz_sem,
                inc=1,
                device_id=pz,
                device_id_type=pl.DeviceIdType.MESH,
            )

            cpa = pltpu.make_async_copy(
                x_hbm.at[pl.ds(fwd, F), :], xv.at[pl.ds(fwd, F), :], cpa_sem
            )
            cpb = pltpu.make_async_copy(
                x_hbm.at[pl.ds(2 * F, d_rows), :],
                xv.at[pl.ds(2 * F, d_rows), :],
                cpb_sem,
            )
            cpc = pltpu.make_async_copy(
                x_hbm.at[pl.ds(kept, F), :], xv.at[pl.ds(kept, F), :], cpc_sem
            )
            cpa.start()
            cpb.start()
            cpc.start()
            pl.semaphore_wait(barrier_sem, 1)

            cpa.wait()
            p1f = []
            for c in range(NF):
                s = own + fwd + c * kf
                out_ref[pl.ds(s, kf), :] = xv[
                    pl.ds(fwd + c * kf, kf), :
                ].astype(jnp.bfloat16)
                r = pltpu.make_async_remote_copy(
                    src_ref=out_ref.at[pl.ds(s, kf), :],
                    dst_ref=out_ref.at[pl.ds(s, kf), :],
                    send_sem=p1f_send.at[c],
                    recv_sem=p1f_recv.at[c],
                    device_id=px,
                    device_id_type=pl.DeviceIdType.MESH,
                )
                r.start()
                p1f.append(r)
            cpb.wait()
            p1d = []
            for c in range(ND):
                s = own + 2 * F + c * kd
                out_ref[pl.ds(s, kd), :] = xv[
                    pl.ds(2 * F + c * kd, kd), :
                ].astype(jnp.bfloat16)
                r = pltpu.make_async_remote_copy(
                    src_ref=out_ref.at[pl.ds(s, kd), :],
                    dst_ref=out_ref.at[pl.ds(s, kd), :],
                    send_sem=p1d_send.at[c],
                    recv_sem=p1d_recv.at[c],
                    device_id=px,
                    device_id_type=pl.DeviceIdType.MESH,
                )
                r.start()
                p1d.append(r)

            cpc.wait()
            out_ref[pl.ds(own + kept, F), :] = xv[
                pl.ds(kept, F), :
            ].astype(jnp.bfloat16)

            pl.semaphore_wait(pz_sem, 1)

            p2 = []
            for c in range(NF):
                p1f[c].wait_recv()
                s = opp + fwd + c * kf
                r = pltpu.make_async_remote_copy(
                    src_ref=out_ref.at[pl.ds(s, kf), :],
                    dst_ref=out_ref.at[pl.ds(s, kf), :],
                    send_sem=p2_send.at[c],
                    recv_sem=p2_recv.at[c],
                    device_id=pz,
                    device_id_type=pl.DeviceIdType.MESH,
                )
                r.start()
                p2.append(r)

            for c in range(ND):
                p1d[c].wait_recv()
            for c in range(NF):
                p2[c].wait_recv()
                p1f[c].wait_send()
                p2[c].wait_send()
            for c in range(ND):
                p1d[c].wait_send()

    return pl.pallas_call(
        body,
        out_shape=jax.ShapeDtypeStruct((2 * m_per, n), jnp.bfloat16),
        in_specs=[pl.BlockSpec(memory_space=pl.ANY)],
        out_specs=pl.BlockSpec(memory_space=pltpu.VMEM),
        scratch_shapes=[
            pltpu.VMEM((m_per, n), x.dtype),
            pltpu.SemaphoreType.DMA,
            pltpu.SemaphoreType.DMA,
            pltpu.SemaphoreType.DMA,
            pltpu.SemaphoreType.DMA((NF,)),
            pltpu.SemaphoreType.DMA((NF,)),
            pltpu.SemaphoreType.DMA((ND,)),
            pltpu.SemaphoreType.DMA((ND,)),
            pltpu.SemaphoreType.DMA((NF,)),
            pltpu.SemaphoreType.DMA((NF,)),
        ],
        compiler_params=pltpu.CompilerParams(collective_id=0),
    )(x)
